# baseline (speedup 1.0000x reference)
import functools

import jax
import jax.numpy as jnp
from jax import lax
from jax.experimental import pallas as pl
from jax.experimental.pallas import tpu as pltpu
from jax.experimental.pallas import tpu_sc as plsc

N_FEAT = 1000000
D = 64
N_POINTS = 819200
L = 16
NC = 2
NS = 16
NW = NC * NS
ROWS_PER_W = N_POINTS // NW
GSLAB = 128
CHUNK = 512
GATHERS = CHUNK // GSLAB
N_CHUNKS = ROWS_PER_W // CHUNK


def _shuffle16(x, idx):
    dnums = lax.GatherDimensionNumbers(
        offset_dims=(), collapsed_slice_dims=(0,), start_index_map=(0,)
    )
    return lax.gather(
        x,
        idx[:, None],
        dnums,
        slice_sizes=(1,),
        mode=lax.GatherScatterMode.PROMISE_IN_BOUNDS,
    )


def _rsqrt_scalar(x):
    i = lax.bitcast_convert_type(x, jnp.int32)
    y = lax.bitcast_convert_type(
        jnp.int32(0x5F3759DF) - (i >> 1), jnp.float32
    )
    for _ in range(3):
        y = y * (1.5 - 0.5 * x * y * y)
    return y


def _body(feat_hbm, v2p_hbm, out_hbm, idx_v, rows_v, sem):
    cid = lax.axis_index("c")
    sid = lax.axis_index("s")
    wid = sid * NC + cid

    def chunk_body(g, carry):
        slab0 = wid * (ROWS_PER_W // GSLAB) + g * GATHERS
        pltpu.sync_copy(v2p_hbm.at[pl.ds(slab0, GATHERS)], idx_v)
        copies = []
        for j in range(GATHERS):
            copies.append(
                pltpu.async_copy(
                    feat_hbm.at[idx_v.at[j]],
                    rows_v.at[pl.ds(j * GSLAB, GSLAB)],
                    sem,
                )
            )
        for cp in copies:
            cp.wait()

        lane = lax.iota(jnp.int32, L)
        perms = [lane ^ k for k in (1, 2, 4, 8)]

        def row_body(r, c):
            v0 = rows_v[r, pl.ds(0, L)]
            v1 = rows_v[r, pl.ds(L, L)]
            v2 = rows_v[r, pl.ds(2 * L, L)]
            v3 = rows_v[r, pl.ds(3 * L, L)]
            ss = v0 * v0 + v1 * v1 + v2 * v2 + v3 * v3
            for p in perms:
                ss = ss + _shuffle16(ss, p)
            inv = _rsqrt_scalar(ss[0])
            rows_v[r, pl.ds(0, L)] = v0 * inv
            rows_v[r, pl.ds(L, L)] = v1 * inv
            rows_v[r, pl.ds(2 * L, L)] = v2 * inv
            rows_v[r, pl.ds(3 * L, L)] = v3 * inv
            return c

        lax.fori_loop(0, CHUNK, row_body, 0)
        row_base = wid * ROWS_PER_W + g * CHUNK
        pltpu.sync_copy(rows_v, out_hbm.at[pl.ds(row_base, CHUNK)])
        return carry

    lax.fori_loop(0, N_CHUNKS, chunk_body, 0)


def kernel(features, v2p_map):
    v2p2 = v2p_map.reshape(N_POINTS // GSLAB, GSLAB)
    mesh = plsc.VectorSubcoreMesh(core_axis_name="c", subcore_axis_name="s")
    run = functools.partial(
        pl.kernel,
        mesh=mesh,
        out_type=jax.ShapeDtypeStruct((N_POINTS, D), jnp.float32),
        scratch_types=[
            pltpu.VMEM((GATHERS, GSLAB), jnp.int32),
            pltpu.VMEM((CHUNK, D), jnp.float32),
            pltpu.SemaphoreType.DMA,
        ],
        compiler_params=pltpu.CompilerParams(use_tc_tiling_on_sc=False),
    )(_body)
    return run(features, v2p2)


# speedup vs baseline: 1.1148x; 1.1148x over previous
import functools

import jax
import jax.numpy as jnp
from jax import lax
from jax.experimental import pallas as pl
from jax.experimental.pallas import tpu as pltpu
from jax.experimental.pallas import tpu_sc as plsc

N_FEAT = 1000000
D = 64
N_POINTS = 819200
L = 16
NC = 2
NS = 16
NW = NC * NS
ROWS_PER_W = N_POINTS // NW
CHUNK = 128
N_CHUNKS = ROWS_PER_W // CHUNK


def _shuffle16(x, idx):
    dnums = lax.GatherDimensionNumbers(
        offset_dims=(), collapsed_slice_dims=(0,), start_index_map=(0,)
    )
    return lax.gather(
        x,
        idx[:, None],
        dnums,
        slice_sizes=(1,),
        mode=lax.GatherScatterMode.PROMISE_IN_BOUNDS,
    )


def _rsqrt_scalar(x):
    i = lax.bitcast_convert_type(x, jnp.int32)
    y = lax.bitcast_convert_type(
        jnp.int32(0x5F3759DF) - (i >> 1), jnp.float32
    )
    hx = 0.5 * x
    for _ in range(2):
        y = y * (1.5 - hx * y * y)
    return y


def _body(feat_hbm, idx_hbm, lo_hbm, out_hbm,
          idx_v, lo_v, rows_v, out_v, gsem, osem):
    cid = lax.axis_index("c")
    sid = lax.axis_index("s")
    wid = sid * NC + cid
    slab0 = wid * N_CHUNKS
    row0 = wid * ROWS_PER_W

    lane = lax.iota(jnp.int32, L)
    perms = [lane ^ k for k in (1, 2, 4, 8)]

    def stage_and_gather(g, b):
        pltpu.sync_copy(idx_hbm.at[pl.ds(slab0 + g, 1)], idx_v.at[b])
        pltpu.sync_copy(lo_hbm.at[pl.ds(slab0 + g, 1)], lo_v.at[b])
        pltpu.async_copy(feat_hbm.at[idx_v.at[b].at[0]], rows_v.at[b], gsem)

    def wait_gather(b):
        pltpu.make_async_copy(
            feat_hbm.at[pl.ds(0, CHUNK)], rows_v.at[b], gsem
        ).wait()

    def wait_store(b):
        pltpu.make_async_copy(
            out_v.at[b], out_hbm.at[pl.ds(0, CHUNK)], osem
        ).wait()

    def normalize_chunk(b):
        def group_body(q, c):
            r0 = q * L
            lov = lo_v[b, 0, pl.ds(r0, L)]
            for i in range(L):
                col = lov[i]
                r = r0 + i
                v0 = rows_v[b, r, pl.ds(col, L)]
                v1 = rows_v[b, r, pl.ds(col + L, L)]
                v2 = rows_v[b, r, pl.ds(col + 2 * L, L)]
                v3 = rows_v[b, r, pl.ds(col + 3 * L, L)]
                ss = v0 * v0 + v1 * v1 + v2 * v2 + v3 * v3
                for p in perms:
                    ss = ss + _shuffle16(ss, p)
                inv = _rsqrt_scalar(ss[0])
                out_v[b, r, pl.ds(0, L)] = v0 * inv
                out_v[b, r, pl.ds(L, L)] = v1 * inv
                out_v[b, r, pl.ds(2 * L, L)] = v2 * inv
                out_v[b, r, pl.ds(3 * L, L)] = v3 * inv
            return c

        lax.fori_loop(0, CHUNK // L, group_body, 0)

    stage_and_gather(0, 0)

    def outer(i, carry):
        g0 = i * 2
        for b in range(2):
            g = g0 + b
            nb = 1 - b

            @pl.when(g + 1 < N_CHUNKS)
            def _():
                stage_and_gather(g + 1, nb)

            wait_gather(b)

            @pl.when(g >= 2)
            def _():
                wait_store(b)

            normalize_chunk(b)
            pltpu.async_copy(
                out_v.at[b], out_hbm.at[pl.ds(row0 + g * CHUNK, CHUNK)], osem
            )
        return carry

    lax.fori_loop(0, N_CHUNKS // 2, outer, 0, unroll=1)

    wait_store(0)
    wait_store(1)


def kernel(features, v2p_map):
    feat2 = features.reshape(N_FEAT // 2, 2 * D)
    idx_hi = (v2p_map >> 1).reshape(N_POINTS // CHUNK, CHUNK)
    lo64 = ((v2p_map & 1) << 6).reshape(N_POINTS // CHUNK, CHUNK)
    mesh = plsc.VectorSubcoreMesh(core_axis_name="c", subcore_axis_name="s")
    run = functools.partial(
        pl.kernel,
        mesh=mesh,
        out_type=jax.ShapeDtypeStruct((N_POINTS, D), jnp.float32),
        scratch_types=[
            pltpu.VMEM((2, 1, CHUNK), jnp.int32),
            pltpu.VMEM((2, 1, CHUNK), jnp.int32),
            pltpu.VMEM((2, CHUNK, 2 * D), jnp.float32),
            pltpu.VMEM((2, CHUNK, D), jnp.float32),
            pltpu.SemaphoreType.DMA,
            pltpu.SemaphoreType.DMA,
        ],
    )(_body)
    return run(feat2, idx_hi, lo64)


# speedup vs baseline: 1.9218x; 1.7238x over previous
import functools

import jax
import jax.numpy as jnp
from jax import lax
from jax.experimental import pallas as pl
from jax.experimental.pallas import tpu as pltpu
from jax.experimental.pallas import tpu_sc as plsc

N_FEAT = 1000000
D = 64
N_POINTS = 819200
L = 16
NC = 2
NS = 16
NW = NC * NS
ROWS_PER_W = N_POINTS // NW
CHUNK = 128
N_CHUNKS = ROWS_PER_W // CHUNK
SUB = 8


def _rsqrt16(x):
    i = plsc.bitcast(x, jnp.int32)
    y = plsc.bitcast(jnp.int32(0x5F3759DF) - (i >> 1), jnp.float32)
    hx = 0.5 * x
    for _ in range(2):
        y = y * (1.5 - hx * y * y)
    return y


def _body(feat_hbm, idx_hbm, out_hbm, idx_v, hi_v, rows_v, out_v, gsem, osem):
    cid = lax.axis_index("c")
    sid = lax.axis_index("s")
    wid = sid * NC + cid
    slab0 = wid * N_CHUNKS
    row0 = wid * ROWS_PER_W

    lane = lax.iota(jnp.int32, L)
    masks = [lane == l for l in range(L)]

    def stage_and_gather(g, b):
        pltpu.sync_copy(idx_hbm.at[pl.ds(slab0 + g, 1)], idx_v.at[b])
        for t in range(CHUNK // L):
            w = idx_v[b, 0, pl.ds(t * L, L)]
            hi_v[b, 0, pl.ds(t * L, L)] = w >> 1
        pltpu.async_copy(feat_hbm.at[hi_v.at[b].at[0]], rows_v.at[b], gsem)

    def wait_gather(b):
        pltpu.make_async_copy(
            feat_hbm.at[pl.ds(0, CHUNK)], rows_v.at[b], gsem
        ).wait()

    def wait_store(b):
        pltpu.make_async_copy(
            out_v.at[b], out_hbm.at[pl.ds(0, CHUNK)], osem
        ).wait()

    def normalize_chunk(b):
        def group_body(q, c):
            r0 = q * L
            idxv = idx_v[b, 0, pl.ds(r0, L)]
            colv = (idxv & 1) << 6
            for h in range(L // SUB):
                tv = jnp.zeros((L,), jnp.float32)
                regs = []
                for i in range(SUB):
                    ln = h * SUB + i
                    r = r0 + ln
                    col = colv[ln]
                    v0 = rows_v[b, r, pl.ds(col, L)]
                    v1 = rows_v[b, r, pl.ds(col + L, L)]
                    v2 = rows_v[b, r, pl.ds(col + 2 * L, L)]
                    v3 = rows_v[b, r, pl.ds(col + 3 * L, L)]
                    ss = v0 * v0 + v1 * v1 + v2 * v2 + v3 * v3
                    tv = jnp.where(masks[ln], jnp.sum(ss), tv)
                    regs.append((v0, v1, v2, v3))
                inv_v = _rsqrt16(tv)
                for i in range(SUB):
                    ln = h * SUB + i
                    r = r0 + ln
                    inv = inv_v[ln]
                    v0, v1, v2, v3 = regs[i]
                    out_v[b, r, pl.ds(0, L)] = v0 * inv
                    out_v[b, r, pl.ds(L, L)] = v1 * inv
                    out_v[b, r, pl.ds(2 * L, L)] = v2 * inv
                    out_v[b, r, pl.ds(3 * L, L)] = v3 * inv
            return c

        lax.fori_loop(0, CHUNK // L, group_body, 0)

    stage_and_gather(0, 0)

    def outer(it, carry):
        g0 = it * 2
        for b in range(2):
            g = g0 + b
            nb = 1 - b

            @pl.when(g + 1 < N_CHUNKS)
            def _():
                stage_and_gather(g + 1, nb)

            wait_gather(b)

            @pl.when(g >= 2)
            def _():
                wait_store(b)

            normalize_chunk(b)
            pltpu.async_copy(
                out_v.at[b], out_hbm.at[pl.ds(row0 + g * CHUNK, CHUNK)], osem
            )
        return carry

    lax.fori_loop(0, N_CHUNKS // 2, outer, 0)

    wait_store(0)
    wait_store(1)


def kernel(features, v2p_map):
    feat2 = features.reshape(N_FEAT // 2, 2 * D)
    idx2 = v2p_map.reshape(N_POINTS // CHUNK, CHUNK)
    mesh = plsc.VectorSubcoreMesh(core_axis_name="c", subcore_axis_name="s")
    run = functools.partial(
        pl.kernel,
        mesh=mesh,
        out_type=jax.ShapeDtypeStruct((N_POINTS, D), jnp.float32),
        scratch_types=[
            pltpu.VMEM((2, 1, CHUNK), jnp.int32),
            pltpu.VMEM((2, 1, CHUNK), jnp.int32),
            pltpu.VMEM((2, CHUNK, 2 * D), jnp.float32),
            pltpu.VMEM((2, CHUNK, D), jnp.float32),
            pltpu.SemaphoreType.DMA,
            pltpu.SemaphoreType.DMA,
        ],
        compiler_params=pltpu.CompilerParams(needs_layout_passes=False),
    )(_body)
    return run(feat2, idx2)
